# baseline (speedup 1.0000x reference)
import jax
import jax.numpy as jnp
from jax.experimental import pallas as pl

N_CLASSES = 1000
EMBED_DIM = 128
BATCH = 16384

_M = 256
_KPAD = 1024


def _tc_body(idx_ref, table_ref, out_ref):
    idxb = idx_ref[0, 0, :]
    onehot = (
        jax.lax.broadcasted_iota(jnp.int32, (_M, _KPAD), 1) == idxb[:, None]
    ).astype(jnp.bfloat16)
    out_ref[...] = jnp.dot(onehot, table_ref[...], preferred_element_type=jnp.float32)


def kernel(class_idx, table):
    nb = BATCH // _M
    idx3 = class_idx.astype(jnp.int32).reshape(nb, 1, _M)
    tpad = (
        jnp.zeros((_KPAD, EMBED_DIM), jnp.bfloat16)
        .at[:N_CLASSES]
        .set(table.astype(jnp.bfloat16))
    )
    out = pl.pallas_call(
        _tc_body,
        grid=(nb,),
        in_specs=[
            pl.BlockSpec((1, 1, _M), lambda i: (i, 0, 0)),
            pl.BlockSpec((_KPAD, EMBED_DIM), lambda i: (0, 0)),
        ],
        out_specs=pl.BlockSpec((_M, EMBED_DIM), lambda i: (i, 0)),
        out_shape=jax.ShapeDtypeStruct((BATCH, EMBED_DIM), jnp.float32),
    )(idx3, tpad)
    return out.reshape(BATCH, 1, EMBED_DIM)
